# Initial kernel scaffold; baseline (speedup 1.0000x reference)
import jax
import jax.numpy as jnp
from jax.experimental import pallas as pl


def kernel(embedding, r_embedding):
    raise NotImplementedError("write your pallas kernel here")


# speedup vs baseline: 2.4936x; 2.4936x over previous
import jax
import jax.numpy as jnp
from jax.experimental import pallas as pl

N = 8192
BR = 512
BC = 2048


def _body(emb_ref, r_ref, val_ref, arg_ref):
    r = pl.program_id(1)

    @pl.when(r == 0)
    def _init():
        val_ref[...] = jnp.zeros_like(val_ref)
        arg_ref[...] = jnp.zeros_like(arg_ref)

    emb = emb_ref[0, :]
    blk = r_ref[...]
    p = blk * emb[:, None]
    m = jnp.max(p, axis=0)
    rows = jax.lax.broadcasted_iota(jnp.int32, p.shape, 0)
    cand = jnp.where(p == m[None, :], rows, N)
    a = jnp.min(cand, axis=0) + r * BR
    cur = val_ref[0, :]
    upd = m > cur
    val_ref[0, :] = jnp.where(upd, m, cur)
    arg_ref[0, :] = jnp.where(upd, a.astype(jnp.float32), arg_ref[0, :])


def kernel(embedding, r_embedding):
    grid = (N // BC, N // BR)
    val, arg = pl.pallas_call(
        _body,
        grid=grid,
        in_specs=[
            pl.BlockSpec((1, BR), lambda c, r: (0, r)),
            pl.BlockSpec((BR, BC), lambda c, r: (r, c)),
        ],
        out_specs=[
            pl.BlockSpec((1, BC), lambda c, r: (0, c)),
            pl.BlockSpec((1, BC), lambda c, r: (0, c)),
        ],
        out_shape=[
            jax.ShapeDtypeStruct((1, N), jnp.float32),
            jax.ShapeDtypeStruct((1, N), jnp.float32),
        ],
    )(embedding, r_embedding)
    return val, arg[0]
